# baseline (speedup 1.0000x reference)
import functools

import jax
import jax.numpy as jnp
from jax import lax
from jax.experimental import pallas as pl
from jax.experimental.pallas import tpu as pltpu
from jax.experimental.pallas import tpu_sc as plsc

_B = 16384
_F = 26
_D = 32
_NW = 32
_BLK = 256
_BPF = _B // _BLK
_NB = _B * _F // _BLK // _NW
_ORB = _BLK * _D // 128
_LANES = 16
_GRP = _BLK // _LANES


def _make_kernel():
    mesh = plsc.VectorSubcoreMesh(core_axis_name="c", subcore_axis_name="s")

    @functools.partial(
        pl.kernel,
        mesh=mesh,
        compiler_params=pltpu.CompilerParams(use_tc_tiling_on_sc=True),
        out_type=jax.ShapeDtypeStruct((_B * _F * _D // 128, 128), jnp.float32),
        scratch_types=[
            pltpu.VMEM((48,), jnp.int32),
            pltpu.VMEM((_BLK,), jnp.int32),
            pltpu.VMEM((_BLK,), jnp.int32),
            pltpu.VMEM((_BLK,), jnp.int32),
            pltpu.VMEM((_BLK,), jnp.int32),
            pltpu.VMEM((_BLK, 128), jnp.float32),
            pltpu.VMEM((_BLK, 128), jnp.float32),
            pltpu.VMEM((_ORB, 128), jnp.float32),
            pltpu.VMEM((_ORB, 128), jnp.float32),
            pltpu.SemaphoreType.DMA,
            pltpu.SemaphoreType.DMA,
            pltpu.SemaphoreType.DMA,
        ],
    )
    def k(catf_hbm, table4_hbm, offs_hbm, out_hbm, offs_v,
          lineA, lineB, qoffA, qoffB, rowsA, rowsB, outA, outB,
          semA, semB, sem_w):
        line = (lineA, lineB)
        qoff = (qoffA, qoffB)
        rows = (rowsA, rowsB)
        outb = (outA, outB)
        sem_g = (semA, semB)
        wid = lax.axis_index("s") * 2 + lax.axis_index("c")
        g0 = wid * _NB

        pltpu.sync_copy(offs_hbm, offs_v)

        def prep(g, s):
            f = g // _BPF
            off = offs_v[pl.ds(f, _LANES)][0]
            pltpu.sync_copy(catf_hbm.at[pl.ds(g * _BLK, _BLK)], line[s])

            def body(i, carry):
                sl = pl.ds(i * _LANES, _LANES)
                r = line[s][sl] + off
                qoff[s][sl] = (r & 3) << 5
                line[s][sl] = r >> 2
                return carry

            lax.fori_loop(0, _GRP, body, None)
            return pltpu.async_copy(table4_hbm.at[line[s]], rows[s], sem_g[s])

        def extract(s):
            def body(gg, carry):
                qv = qoff[s][pl.ds(gg * _LANES, _LANES)]
                for l in range(_LANES):
                    i = gg * _LANES + l
                    orow = gg * 4 + l // 4
                    ocol = (l % 4) * _D
                    o = qv[l]
                    for h in range(2):
                        outb[s][orow, pl.ds(ocol + h * _LANES, _LANES)] = (
                            rows[s][i, pl.ds(o + h * _LANES, _LANES)])
                return carry

            lax.fori_loop(0, _GRP, body, None)

        def pair(t, carry):
            gA = g0 + t * 2
            gB = gA + 1
            ghA = prep(gA, 0)
            ghB = prep(gB, 1)
            ghA.wait()
            extract(0)
            whA = pltpu.async_copy(outA, out_hbm.at[pl.ds(gA * _ORB, _ORB)],
                                   sem_w)
            ghB.wait()
            extract(1)
            whB = pltpu.async_copy(outB, out_hbm.at[pl.ds(gB * _ORB, _ORB)],
                                   sem_w)
            whA.wait()
            whB.wait()
            return carry

        lax.fori_loop(0, _NB // 2, pair, None)

    return k


_lookup = _make_kernel()


def kernel(categorical_inputs, embedding_weight, offsets):
    catf = categorical_inputs.T.reshape(-1).astype(jnp.int32)
    table4 = embedding_weight.reshape(_F * 100000 * _D // 128, 128)
    offs_pad = jnp.pad(offsets[:-1].astype(jnp.int32), (0, 48 - _F))
    out = _lookup(catf, table4, offs_pad)
    return out.reshape(_F, _B, _D).transpose(1, 0, 2)


# speedup vs baseline: 1.1128x; 1.1128x over previous
import functools

import jax
import jax.numpy as jnp
from jax import lax
from jax.experimental import pallas as pl
from jax.experimental.pallas import tpu as pltpu
from jax.experimental.pallas import tpu_sc as plsc

_B = 16384
_F = 26
_D = 32
_NW = 32
_BLK = 1024
_BPF = _B // _BLK
_NB = _B * _F // _BLK // _NW
_LANES = 16
_NBUF = 3


def _make_kernel():
    mesh = plsc.VectorSubcoreMesh(core_axis_name="c", subcore_axis_name="s")

    @functools.partial(
        pl.kernel,
        mesh=mesh,
        compiler_params=pltpu.CompilerParams(use_tc_tiling_on_sc=False),
        out_type=jax.ShapeDtypeStruct((_B * _F, _D), jnp.float32),
        scratch_types=[
            pltpu.VMEM((48,), jnp.int32),
            pltpu.VMEM((_NB, _BLK), jnp.int32),
            pltpu.VMEM((_NBUF, _BLK, _D), jnp.float32),
            pltpu.SemaphoreType.DMA,
            pltpu.SemaphoreType.DMA,
            pltpu.SemaphoreType.DMA,
            pltpu.SemaphoreType.DMA,
        ],
    )
    def k(catf_hbm, table_hbm, offs_hbm, out_hbm, offs_v, idx_v, rows_v,
          sem_g0, sem_g1, sem_g2, sem_w):
        sem_g = (sem_g0, sem_g1, sem_g2)
        wid = lax.axis_index("s") * 2 + lax.axis_index("c")
        g0 = wid * _NB

        pltpu.sync_copy(offs_hbm, offs_v)

        def prep(j):
            g = g0 + j
            f = g // _BPF
            pltpu.sync_copy(catf_hbm.at[pl.ds(g * _BLK, _BLK)], idx_v.at[j])
            off = offs_v[pl.ds(f, _LANES)][0]

            def body(i, carry):
                s = i * _LANES
                idx_v[j, pl.ds(s, _LANES)] = idx_v[j, pl.ds(s, _LANES)] + off
                return carry

            lax.fori_loop(0, _BLK // _LANES, body, None)

        def fire(b):
            return pltpu.async_copy(table_hbm.at[idx_v.at[b]],
                                    rows_v.at[b % _NBUF], sem_g[b % _NBUF])

        gh = [None] * _NB
        wh = [None] * _NB
        for b in range(2):
            prep(b)
            gh[b] = fire(b)
        for b in range(_NB):
            if b + 2 < _NB:
                if b >= 1:
                    wh[b - 1].wait()
                prep(b + 2)
                gh[b + 2] = fire(b + 2)
            gh[b].wait()
            wh[b] = pltpu.async_copy(
                rows_v.at[b % _NBUF],
                out_hbm.at[pl.ds((g0 + b) * _BLK, _BLK)], sem_w)
        wh[_NB - 2].wait()
        wh[_NB - 1].wait()

    return k


_lookup = _make_kernel()


def kernel(categorical_inputs, embedding_weight, offsets):
    catf = categorical_inputs.T.reshape(-1).astype(jnp.int32)
    offs_pad = jnp.pad(offsets[:-1].astype(jnp.int32), (0, 48 - _F))
    out = _lookup(catf, embedding_weight, offs_pad)
    return out.reshape(_F, _B, _D).transpose(1, 0, 2)
